# baseline (speedup 1.0000x reference)
import functools

import jax
import jax.numpy as jnp
from jax import lax
from jax.experimental import pallas as pl
from jax.experimental.pallas import tpu as pltpu
from jax.experimental.pallas import tpu_sc as plsc

N_NODES = 10000
N_EDGES = 320000
D_FEAT = 128
NUM_CLASSES = 2

NP = 10240
NR = NP // 128
E_PER_TILE = N_EDGES // 32
E_PER_SCTILE = N_EDGES // 16


def _z_body(x_ref, w_ref, o_ref):
    res = lax.dot_general(
        w_ref[...], x_ref[...], (((1,), (1,)), ((), ())),
        preferred_element_type=jnp.float32)
    o_ref[...] = res.reshape(8, 16, 128)


def _project(x, w_pad):
    blk = 2048
    return pl.pallas_call(
        _z_body,
        grid=(NP // blk,),
        in_specs=[
            pl.BlockSpec((blk, D_FEAT), lambda i: (i, 0)),
            pl.BlockSpec((8, D_FEAT), lambda i: (0, 0)),
        ],
        out_specs=pl.BlockSpec((8, 16, 128), lambda i: (0, i, 0)),
        out_shape=jax.ShapeDtypeStruct((8, NR, 128), jnp.float32),
    )(x, w_pad)


def _rsqrt16(v):
    bits = lax.bitcast_convert_type(v, jnp.int32)
    y = lax.bitcast_convert_type(jnp.int32(0x5F3759DF) - (bits >> 1), jnp.float32)
    h = v * jnp.float32(-0.5)
    y = y * (jnp.float32(1.5) + h * y * y)
    y = y * (jnp.float32(1.5) + h * y * y)
    y = y * (jnp.float32(1.5) + h * y * y)
    return y


_ZERO16 = functools.partial(jnp.zeros, (16,), jnp.float32)
_ONE16 = functools.partial(jnp.full, (16,), 1.0, jnp.float32)


def _zero_rows(ref):
    def zrow(i, _):
        for l in range(8):
            ref[i, pl.ds(l * 16, 16)] = _ZERO16()
        return 0
    lax.fori_loop(0, NR, zrow, 0)


def _sc_body(eif_hbm, z_hbm, acc_out, dis_out,
             srcv, dstv, dis2, z0, z1, acc_a, acc_b, rowidx, zer, zsem,
             hist_sh):
    c = lax.axis_index("c")
    s = lax.axis_index("s")
    wid = c * 16 + s

    zd0 = pltpu.async_copy(z_hbm.at[0], z0, zsem)
    zd1 = pltpu.async_copy(z_hbm.at[1], z1, zsem)

    def irow(i, _):
        rowidx[pl.ds(i * 16, 16)] = i * 16 + lax.iota(jnp.int32, 16)
        return 0
    lax.fori_loop(0, NR // 16, irow, 0)

    @pl.when(s == 0)
    def _():
        _zero_rows(zer)
        pltpu.sync_copy(zer, hist_sh)

    _zero_rows(acc_a)
    plsc.subcore_barrier()

    with jax.named_scope("p1_stage"):
        dbase = N_EDGES + s * E_PER_SCTILE
        pltpu.sync_copy(eif_hbm.at[pl.ds(dbase, E_PER_TILE)], srcv)
        pltpu.sync_copy(eif_hbm.at[pl.ds(dbase + E_PER_TILE, E_PER_TILE)], dstv)

    with jax.named_scope("p1_deg"):
        def deg_chunk(j, _):
            for l in range(5):
                o = (j * 5 + l) * 16
                for buf in (srcv, dstv):
                    ii = buf[pl.ds(o, 16)]
                    plsc.addupdate_scatter(acc_a, [ii >> 7, ii & 127], _ONE16())
            return 0
        lax.fori_loop(0, E_PER_TILE // 80, deg_chunk, 0)

    with jax.named_scope("p1_reduce"):
        pltpu.sync_copy(acc_a, hist_sh.at[rowidx], add=True)
        plsc.subcore_barrier()

    with jax.named_scope("p2_bcast"):
        pltpu.sync_copy(hist_sh, dis2)
        zd0.wait()
        zd1.wait()

    with jax.named_scope("p2_dis"):
        def dis_row(i, _):
            for l in range(8):
                sl16 = pl.ds(l * 16, 16)
                v = dis2[i, sl16] + jnp.float32(1.0)
                y = _rsqrt16(v)
                dis2[i, sl16] = y
                z0[i, sl16] = z0[i, sl16] * y
                z1[i, sl16] = z1[i, sl16] * y
            return 0
        lax.fori_loop(0, NR, dis_row, 0)

        @pl.when(jnp.logical_and(c == 0, s == 0))
        def _():
            pltpu.sync_copy(dis2, dis_out)

    with jax.named_scope("p3_zero"):
        _zero_rows(acc_a)
        _zero_rows(acc_b)

    with jax.named_scope("p3_stage"):
        pltpu.sync_copy(eif_hbm.at[pl.ds(wid * E_PER_TILE, E_PER_TILE)], srcv)
        pltpu.sync_copy(
            eif_hbm.at[pl.ds(N_EDGES + wid * E_PER_TILE, E_PER_TILE)], dstv)

    with jax.named_scope("p3_msg"):
        def msg_chunk(j, _):
            for l in range(5):
                o = (j * 5 + l) * 16
                ss = srcv[pl.ds(o, 16)]
                dd = dstv[pl.ds(o, 16)]
                dr = dd >> 7
                dl = dd & 127
                plsc.addupdate_scatter(
                    acc_a, [dr, dl], plsc.load_gather(z0, [ss >> 7, ss & 127]))
                plsc.addupdate_scatter(
                    acc_b, [dr, dl], plsc.load_gather(z1, [ss >> 7, ss & 127]))
            return 0
        lax.fori_loop(0, E_PER_TILE // 80, msg_chunk, 0)

    with jax.named_scope("p4_out"):
        pltpu.sync_copy(acc_a, acc_out.at[pl.ds(wid * NR, NR)])
        pltpu.sync_copy(acc_b, acc_out.at[pl.ds((32 + wid) * NR, NR)])


_sc_kernel = functools.partial(
    pl.kernel,
    out_type=(
        jax.ShapeDtypeStruct((64 * NR, 128), jnp.float32),
        jax.ShapeDtypeStruct((NR, 128), jnp.float32),
    ),
    mesh=plsc.VectorSubcoreMesh(
        core_axis_name="c", subcore_axis_name="s", num_cores=2, num_subcores=16),
    compiler_params=pltpu.CompilerParams(needs_layout_passes=False),
    scratch_types=[
        pltpu.VMEM((E_PER_TILE,), jnp.int32),
        pltpu.VMEM((E_PER_TILE,), jnp.int32),
        pltpu.VMEM((NR, 128), jnp.float32),
        pltpu.VMEM((NR, 128), jnp.float32),
        pltpu.VMEM((NR, 128), jnp.float32),
        pltpu.VMEM((NR, 128), jnp.float32),
        pltpu.VMEM((NR, 128), jnp.float32),
        pltpu.VMEM((NR,), jnp.int32),
        pltpu.VMEM((NR, 128), jnp.float32),
        pltpu.SemaphoreType.DMA,
        pltpu.VMEM_SHARED((NR, 128), jnp.float32),
    ],
)(_sc_body)


def _fin_body(acc_ref, dis_ref, z_ref, b_ref, o_ref):
    a = acc_ref[...]
    p0 = a[0, 0]
    p1 = a[1, 0]
    for t in range(1, 32):
        p0 = p0 + a[0, t]
        p1 = p1 + a[1, t]
    d = dis_ref[...]
    d2 = d * d
    zz = z_ref[...]
    bb = b_ref[...]
    o0 = d * p0 + d2 * zz[0] + bb[0, 0]
    o1 = d * p1 + d2 * zz[1] + bb[1, 0]
    m = jnp.maximum(o0, o1)
    lse = m + jnp.log(jnp.exp(o0 - m) + jnp.exp(o1 - m))
    o_ref[...] = jnp.stack([o0 - lse, o1 - lse], 0)


def _finalize(acc, dis, zT, b2):
    return pl.pallas_call(
        _fin_body,
        grid=(NR // 16,),
        in_specs=[
            pl.BlockSpec((2, 32, 16, 128), lambda i: (0, 0, i, 0)),
            pl.BlockSpec((16, 128), lambda i: (i, 0)),
            pl.BlockSpec((8, 16, 128), lambda i: (0, i, 0)),
            pl.BlockSpec((2, 128), lambda i: (0, 0)),
        ],
        out_specs=pl.BlockSpec((2, 16, 128), lambda i: (0, i, 0)),
        out_shape=jax.ShapeDtypeStruct((2, NR, 128), jnp.float32),
    )(acc, dis, zT, b2)


def kernel(x, edge_index, W, b):
    eif = edge_index.astype(jnp.int32).reshape(2 * N_EDGES)
    w_pad = jnp.pad(W, ((0, 8 - NUM_CLASSES), (0, 0)))
    zT = _project(x, w_pad)
    acc, dis = _sc_kernel(eif, zT)
    b2 = jnp.broadcast_to(b[:, None], (NUM_CLASSES, 128))
    out = _finalize(acc.reshape(2, 32, NR, 128), dis, zT, b2)
    return out.reshape(2, NP)[:, :N_NODES].T


# speedup vs baseline: 1.0427x; 1.0427x over previous
import functools

import jax
import jax.numpy as jnp
from jax import lax
from jax.experimental import pallas as pl
from jax.experimental.pallas import tpu as pltpu
from jax.experimental.pallas import tpu_sc as plsc

N_NODES = 10000
N_EDGES = 320000
D_FEAT = 128
NUM_CLASSES = 2

NP = 10240
NR = NP // 128
E_PER_TILE = N_EDGES // 32
E_PER_SCTILE = N_EDGES // 16


def _z_body(x_ref, w_ref, o_ref):
    res = lax.dot_general(
        w_ref[...], x_ref[...], (((1,), (1,)), ((), ())),
        preferred_element_type=jnp.float32)
    o_ref[...] = res.reshape(8, 16, 128)


def _project(x, w_pad):
    blk = 2048
    return pl.pallas_call(
        _z_body,
        grid=(NP // blk,),
        in_specs=[
            pl.BlockSpec((blk, D_FEAT), lambda i: (i, 0)),
            pl.BlockSpec((8, D_FEAT), lambda i: (0, 0)),
        ],
        out_specs=pl.BlockSpec((8, 16, 128), lambda i: (0, i, 0)),
        out_shape=jax.ShapeDtypeStruct((8, NR, 128), jnp.float32),
    )(x, w_pad)


def _rsqrt16(v):
    bits = lax.bitcast_convert_type(v, jnp.int32)
    y = lax.bitcast_convert_type(jnp.int32(0x5F3759DF) - (bits >> 1), jnp.float32)
    h = v * jnp.float32(-0.5)
    y = y * (jnp.float32(1.5) + h * y * y)
    y = y * (jnp.float32(1.5) + h * y * y)
    y = y * (jnp.float32(1.5) + h * y * y)
    return y


_ZERO16 = functools.partial(jnp.zeros, (16,), jnp.float32)
_ONE16 = functools.partial(jnp.full, (16,), 1.0, jnp.float32)


def _zero_rows(ref):
    def zrow(i, _):
        for l in range(8):
            ref[i, pl.ds(l * 16, 16)] = _ZERO16()
        return 0
    lax.fori_loop(0, NR, zrow, 0)


def _sc_body(eif_hbm, z_hbm, acc_out, dis_out,
             srcv, dstv, dis2, z0, z1, acc_a, acc_b, rowidx, zer, zsem, esem,
             hist_sh):
    c = lax.axis_index("c")
    s = lax.axis_index("s")
    wid = c * 16 + s

    zd0 = pltpu.async_copy(z_hbm.at[0], z0, zsem)
    zd1 = pltpu.async_copy(z_hbm.at[1], z1, zsem)
    dbase = N_EDGES + s * E_PER_SCTILE
    e1a = pltpu.async_copy(eif_hbm.at[pl.ds(dbase, E_PER_TILE)], srcv, esem)
    e1b = pltpu.async_copy(
        eif_hbm.at[pl.ds(dbase + E_PER_TILE, E_PER_TILE)], dstv, esem)

    def irow(i, _):
        rowidx[pl.ds(i * 16, 16)] = i * 16 + lax.iota(jnp.int32, 16)
        return 0
    lax.fori_loop(0, NR // 16, irow, 0)

    @pl.when(s == 0)
    def _():
        _zero_rows(zer)
        pltpu.sync_copy(zer, hist_sh)

    _zero_rows(acc_a)
    plsc.subcore_barrier()

    with jax.named_scope("p1_stage"):
        e1a.wait()
        e1b.wait()

    with jax.named_scope("p1_deg"):
        def deg_chunk(j, _):
            for l in range(5):
                o = (j * 5 + l) * 16
                for buf in (srcv, dstv):
                    ii = buf[pl.ds(o, 16)]
                    plsc.addupdate_scatter(acc_a, [ii >> 7, ii & 127], _ONE16())
            return 0
        lax.fori_loop(0, E_PER_TILE // 80, deg_chunk, 0)

    e3a = pltpu.async_copy(
        eif_hbm.at[pl.ds(wid * E_PER_TILE, E_PER_TILE)], srcv, esem)
    e3b = pltpu.async_copy(
        eif_hbm.at[pl.ds(N_EDGES + wid * E_PER_TILE, E_PER_TILE)], dstv, esem)

    with jax.named_scope("p1_reduce"):
        pltpu.sync_copy(acc_a, hist_sh.at[rowidx], add=True)
        plsc.subcore_barrier()

    with jax.named_scope("p2_bcast"):
        pltpu.sync_copy(hist_sh, dis2)
        zd0.wait()
        zd1.wait()

    with jax.named_scope("p2_dis"):
        def dis_row(i, _):
            for l in range(8):
                sl16 = pl.ds(l * 16, 16)
                v = dis2[i, sl16] + jnp.float32(1.0)
                y = _rsqrt16(v)
                dis2[i, sl16] = y
                z0[i, sl16] = z0[i, sl16] * y
                z1[i, sl16] = z1[i, sl16] * y
            return 0
        lax.fori_loop(0, NR, dis_row, 0)

        @pl.when(jnp.logical_and(c == 0, s == 0))
        def _():
            pltpu.sync_copy(dis2, dis_out)

    with jax.named_scope("p3_zero"):
        _zero_rows(acc_a)
        _zero_rows(acc_b)

    with jax.named_scope("p3_stage"):
        e3a.wait()
        e3b.wait()

    with jax.named_scope("p3_msg"):
        def msg_chunk(j, _):
            for l in range(5):
                o = (j * 5 + l) * 16
                ss = srcv[pl.ds(o, 16)]
                dd = dstv[pl.ds(o, 16)]
                dr = dd >> 7
                dl = dd & 127
                plsc.addupdate_scatter(
                    acc_a, [dr, dl], plsc.load_gather(z0, [ss >> 7, ss & 127]))
                plsc.addupdate_scatter(
                    acc_b, [dr, dl], plsc.load_gather(z1, [ss >> 7, ss & 127]))
            return 0
        lax.fori_loop(0, E_PER_TILE // 80, msg_chunk, 0)

    with jax.named_scope("p4_out"):
        pltpu.sync_copy(acc_a, acc_out.at[pl.ds(wid * NR, NR)])
        pltpu.sync_copy(acc_b, acc_out.at[pl.ds((32 + wid) * NR, NR)])


_sc_kernel = functools.partial(
    pl.kernel,
    out_type=(
        jax.ShapeDtypeStruct((64 * NR, 128), jnp.float32),
        jax.ShapeDtypeStruct((NR, 128), jnp.float32),
    ),
    mesh=plsc.VectorSubcoreMesh(
        core_axis_name="c", subcore_axis_name="s", num_cores=2, num_subcores=16),
    compiler_params=pltpu.CompilerParams(needs_layout_passes=False),
    scratch_types=[
        pltpu.VMEM((E_PER_TILE,), jnp.int32),
        pltpu.VMEM((E_PER_TILE,), jnp.int32),
        pltpu.VMEM((NR, 128), jnp.float32),
        pltpu.VMEM((NR, 128), jnp.float32),
        pltpu.VMEM((NR, 128), jnp.float32),
        pltpu.VMEM((NR, 128), jnp.float32),
        pltpu.VMEM((NR, 128), jnp.float32),
        pltpu.VMEM((NR,), jnp.int32),
        pltpu.VMEM((NR, 128), jnp.float32),
        pltpu.SemaphoreType.DMA,
        pltpu.SemaphoreType.DMA,
        pltpu.VMEM_SHARED((NR, 128), jnp.float32),
    ],
)(_sc_body)


def _fin_body(acc_ref, dis_ref, z_ref, b_ref, o_ref):
    a = acc_ref[...]
    p0 = a[0, 0]
    p1 = a[1, 0]
    for t in range(1, 32):
        p0 = p0 + a[0, t]
        p1 = p1 + a[1, t]
    d = dis_ref[...]
    d2 = d * d
    zz = z_ref[...]
    bb = b_ref[...]
    o0 = d * p0 + d2 * zz[0] + bb[0, 0]
    o1 = d * p1 + d2 * zz[1] + bb[1, 0]
    m = jnp.maximum(o0, o1)
    lse = m + jnp.log(jnp.exp(o0 - m) + jnp.exp(o1 - m))
    o_ref[...] = jnp.stack([o0 - lse, o1 - lse], 0)


def _finalize(acc, dis, zT, b2):
    return pl.pallas_call(
        _fin_body,
        grid=(NR // 16,),
        in_specs=[
            pl.BlockSpec((2, 32, 16, 128), lambda i: (0, 0, i, 0)),
            pl.BlockSpec((16, 128), lambda i: (i, 0)),
            pl.BlockSpec((8, 16, 128), lambda i: (0, i, 0)),
            pl.BlockSpec((2, 128), lambda i: (0, 0)),
        ],
        out_specs=pl.BlockSpec((2, 16, 128), lambda i: (0, i, 0)),
        out_shape=jax.ShapeDtypeStruct((2, NR, 128), jnp.float32),
    )(acc, dis, zT, b2)


def kernel(x, edge_index, W, b):
    eif = edge_index.astype(jnp.int32).reshape(2 * N_EDGES)
    w_pad = jnp.pad(W, ((0, 8 - NUM_CLASSES), (0, 0)))
    zT = _project(x, w_pad)
    acc, dis = _sc_kernel(eif, zT)
    b2 = jnp.broadcast_to(b[:, None], (NUM_CLASSES, 128))
    out = _finalize(acc.reshape(2, 32, NR, 128), dis, zT, b2)
    return out.reshape(2, NP)[:, :N_NODES].T


# speedup vs baseline: 1.1027x; 1.0576x over previous
import functools

import jax
import jax.numpy as jnp
from jax import lax
from jax.experimental import pallas as pl
from jax.experimental.pallas import tpu as pltpu
from jax.experimental.pallas import tpu_sc as plsc

N_NODES = 10000
N_EDGES = 320000
D_FEAT = 128
NUM_CLASSES = 2

NP = 10240
NR = NP // 128
E_PER_TILE = N_EDGES // 32
E_PER_SCTILE = N_EDGES // 16


def _z_body(x_ref, w_ref, o_ref):
    res = lax.dot_general(
        w_ref[...], x_ref[...], (((1,), (1,)), ((), ())),
        preferred_element_type=jnp.float32)
    o_ref[...] = res.reshape(8, 16, 128)


def _project(x, w_pad):
    blk = 2048
    return pl.pallas_call(
        _z_body,
        grid=(NP // blk,),
        in_specs=[
            pl.BlockSpec((blk, D_FEAT), lambda i: (i, 0)),
            pl.BlockSpec((8, D_FEAT), lambda i: (0, 0)),
        ],
        out_specs=pl.BlockSpec((8, 16, 128), lambda i: (0, i, 0)),
        out_shape=jax.ShapeDtypeStruct((8, NR, 128), jnp.float32),
    )(x, w_pad)


def _rsqrt16(v):
    bits = lax.bitcast_convert_type(v, jnp.int32)
    y = lax.bitcast_convert_type(jnp.int32(0x5F3759DF) - (bits >> 1), jnp.float32)
    h = v * jnp.float32(-0.5)
    y = y * (jnp.float32(1.5) + h * y * y)
    y = y * (jnp.float32(1.5) + h * y * y)
    y = y * (jnp.float32(1.5) + h * y * y)
    return y


_ZERO16 = functools.partial(jnp.zeros, (16,), jnp.float32)
_ONE16 = functools.partial(jnp.full, (16,), 1.0, jnp.float32)


def _zero_rows(ref):
    def zrow(i, _):
        for l in range(8):
            ref[i, pl.ds(l * 16, 16)] = _ZERO16()
        return 0
    lax.fori_loop(0, NR, zrow, 0)


def _sc_body(eif_hbm, z_hbm, acc_out, dis_out,
             srcv, dstv, dis2, z0, z1, acc_a, acc_b, rowidx, zer, zsem, esem,
             hist_sh):
    c = lax.axis_index("c")
    s = lax.axis_index("s")
    wid = c * 16 + s

    dbase = N_EDGES + s * E_PER_SCTILE
    e1a = pltpu.async_copy(eif_hbm.at[pl.ds(dbase, E_PER_TILE)], srcv, esem)
    e1b = pltpu.async_copy(
        eif_hbm.at[pl.ds(dbase + E_PER_TILE, E_PER_TILE)], dstv, esem)

    def irow(i, _):
        rowidx[pl.ds(i * 16, 16)] = i * 16 + lax.iota(jnp.int32, 16)
        return 0
    lax.fori_loop(0, NR // 16, irow, 0)

    @pl.when(s == 0)
    def _():
        _zero_rows(zer)
        pltpu.sync_copy(zer, hist_sh)

    _zero_rows(acc_a)
    plsc.subcore_barrier()

    with jax.named_scope("p1_stage"):
        e1a.wait()
        e1b.wait()

    zd0 = pltpu.async_copy(z_hbm.at[0], z0, zsem)
    zd1 = pltpu.async_copy(z_hbm.at[1], z1, zsem)

    with jax.named_scope("p1_deg"):
        def deg_chunk(j, _):
            for l in range(5):
                o = (j * 5 + l) * 16
                for buf in (srcv, dstv):
                    ii = buf[pl.ds(o, 16)]
                    plsc.addupdate_scatter(acc_a, [ii >> 7, ii & 127], _ONE16())
            return 0
        lax.fori_loop(0, E_PER_TILE // 80, deg_chunk, 0)

    e3a = pltpu.async_copy(
        eif_hbm.at[pl.ds(wid * E_PER_TILE, E_PER_TILE)], srcv, esem)
    e3b = pltpu.async_copy(
        eif_hbm.at[pl.ds(N_EDGES + wid * E_PER_TILE, E_PER_TILE)], dstv, esem)

    with jax.named_scope("p1_reduce"):
        pltpu.sync_copy(acc_a, hist_sh.at[rowidx], add=True)
        plsc.subcore_barrier()

    with jax.named_scope("p2_bcast"):
        pltpu.sync_copy(hist_sh, dis2)
        zd0.wait()
        zd1.wait()

    with jax.named_scope("p2_dis"):
        def dis_row(i, _):
            for l in range(8):
                sl16 = pl.ds(l * 16, 16)
                v = dis2[i, sl16] + jnp.float32(1.0)
                y = _rsqrt16(v)
                dis2[i, sl16] = y
                z0[i, sl16] = z0[i, sl16] * y
                z1[i, sl16] = z1[i, sl16] * y
            return 0
        lax.fori_loop(0, NR, dis_row, 0)

        @pl.when(jnp.logical_and(c == 0, s == 0))
        def _():
            pltpu.sync_copy(dis2, dis_out)

    with jax.named_scope("p3_zero"):
        _zero_rows(acc_a)
        _zero_rows(acc_b)

    with jax.named_scope("p3_stage"):
        e3a.wait()
        e3b.wait()

    with jax.named_scope("p3_msg"):
        def msg_chunk(j, _):
            for l in range(5):
                o = (j * 5 + l) * 16
                ss = srcv[pl.ds(o, 16)]
                dd = dstv[pl.ds(o, 16)]
                dr = dd >> 7
                dl = dd & 127
                plsc.addupdate_scatter(
                    acc_a, [dr, dl], plsc.load_gather(z0, [ss >> 7, ss & 127]))
                plsc.addupdate_scatter(
                    acc_b, [dr, dl], plsc.load_gather(z1, [ss >> 7, ss & 127]))
            return 0
        lax.fori_loop(0, E_PER_TILE // 80, msg_chunk, 0)

    with jax.named_scope("p4_out"):
        pltpu.sync_copy(acc_a, acc_out.at[pl.ds(wid * NR, NR)])
        pltpu.sync_copy(acc_b, acc_out.at[pl.ds((32 + wid) * NR, NR)])


_sc_kernel = functools.partial(
    pl.kernel,
    out_type=(
        jax.ShapeDtypeStruct((64 * NR, 128), jnp.float32),
        jax.ShapeDtypeStruct((NR, 128), jnp.float32),
    ),
    mesh=plsc.VectorSubcoreMesh(
        core_axis_name="c", subcore_axis_name="s", num_cores=2, num_subcores=16),
    compiler_params=pltpu.CompilerParams(needs_layout_passes=False),
    scratch_types=[
        pltpu.VMEM((E_PER_TILE,), jnp.int32),
        pltpu.VMEM((E_PER_TILE,), jnp.int32),
        pltpu.VMEM((NR, 128), jnp.float32),
        pltpu.VMEM((NR, 128), jnp.float32),
        pltpu.VMEM((NR, 128), jnp.float32),
        pltpu.VMEM((NR, 128), jnp.float32),
        pltpu.VMEM((NR, 128), jnp.float32),
        pltpu.VMEM((NR,), jnp.int32),
        pltpu.VMEM((NR, 128), jnp.float32),
        pltpu.SemaphoreType.DMA,
        pltpu.SemaphoreType.DMA,
        pltpu.VMEM_SHARED((NR, 128), jnp.float32),
    ],
)(_sc_body)


def _fin_body(acc_ref, dis_ref, z_ref, b_ref, o_ref):
    a = acc_ref[...]
    p0 = a[0, 0]
    p1 = a[1, 0]
    for t in range(1, 32):
        p0 = p0 + a[0, t]
        p1 = p1 + a[1, t]
    d = dis_ref[...]
    d2 = d * d
    zz = z_ref[...]
    bb = b_ref[...]
    o0 = d * p0 + d2 * zz[0] + bb[0, 0]
    o1 = d * p1 + d2 * zz[1] + bb[1, 0]
    m = jnp.maximum(o0, o1)
    lse = m + jnp.log(jnp.exp(o0 - m) + jnp.exp(o1 - m))
    o_ref[...] = jnp.stack([o0 - lse, o1 - lse], 0)


def _finalize(acc, dis, zT, b2):
    return pl.pallas_call(
        _fin_body,
        grid=(NR // 16,),
        in_specs=[
            pl.BlockSpec((2, 32, 16, 128), lambda i: (0, 0, i, 0)),
            pl.BlockSpec((16, 128), lambda i: (i, 0)),
            pl.BlockSpec((8, 16, 128), lambda i: (0, i, 0)),
            pl.BlockSpec((2, 128), lambda i: (0, 0)),
        ],
        out_specs=pl.BlockSpec((2, 16, 128), lambda i: (0, i, 0)),
        out_shape=jax.ShapeDtypeStruct((2, NR, 128), jnp.float32),
    )(acc, dis, zT, b2)


def kernel(x, edge_index, W, b):
    eif = edge_index.astype(jnp.int32).reshape(2 * N_EDGES)
    w_pad = jnp.pad(W, ((0, 8 - NUM_CLASSES), (0, 0)))
    zT = _project(x, w_pad)
    acc, dis = _sc_kernel(eif, zT)
    b2 = jnp.broadcast_to(b[:, None], (NUM_CLASSES, 128))
    out = _finalize(acc.reshape(2, 32, NR, 128), dis, zT, b2)
    return out.reshape(2, NP)[:, :N_NODES].T
